# baseline (speedup 1.0000x reference)
import functools

import jax
import jax.numpy as jnp
from jax import lax
from jax.experimental import pallas as pl
from jax.experimental.pallas import tpu as pltpu
from jax.experimental.pallas import tpu_sc as plsc

B = 4096
L = 200
VOCAB = 1000000
EMBED_DIM = 64

_info = plsc.get_sparse_core_info()
_NC = _info.num_cores
_NS = _info.num_subcores
_NW = _NC * _NS

_N = B * L
_PER_W = _N // _NW
_CHUNK = 512
_NCHUNK = _PER_W // _CHUNK


def _gather_kernel(idx_hbm, table_hbm, out_hbm, idx_v, rows_v, sem):
    wid = lax.axis_index("s") * _NC + lax.axis_index("c")
    base = wid * _PER_W

    def body(c, carry):
        start = base + c * _CHUNK
        pltpu.sync_copy(idx_hbm.at[pl.ds(start, _CHUNK)], idx_v)
        pltpu.async_copy(table_hbm.at[idx_v], rows_v, sem).wait()
        pltpu.sync_copy(rows_v, out_hbm.at[pl.ds(start, _CHUNK)])
        return carry

    lax.fori_loop(0, _NCHUNK, body, 0)


@jax.jit
def kernel(tokens, word_embed_weight):
    idx = tokens.reshape(_N).astype(jnp.int32)
    mesh = plsc.VectorSubcoreMesh(core_axis_name="c", subcore_axis_name="s")
    out = pl.kernel(
        _gather_kernel,
        out_type=jax.ShapeDtypeStruct((_N, EMBED_DIM), jnp.float32),
        mesh=mesh,
        scratch_types=[
            pltpu.VMEM((_CHUNK,), jnp.int32),
            pltpu.VMEM((_CHUNK, EMBED_DIM), jnp.float32),
            pltpu.SemaphoreType.DMA,
        ],
        compiler_params=pltpu.CompilerParams(use_tc_tiling_on_sc=False),
    )(idx, word_embed_weight)
    return out.reshape(B, L, EMBED_DIM)


# speedup vs baseline: 1.0404x; 1.0404x over previous
import jax
import jax.numpy as jnp
from jax import lax
from jax.experimental import pallas as pl
from jax.experimental.pallas import tpu as pltpu
from jax.experimental.pallas import tpu_sc as plsc

B = 4096
L = 200
VOCAB = 1000000
EMBED_DIM = 64

_info = plsc.get_sparse_core_info()
_NC = _info.num_cores
_NS = _info.num_subcores
_NW = _NC * _NS

_N = B * L
_PER_W = _N // _NW
_CHUNK = 256
_NBUF = 4
_NCHUNK = _PER_W // _CHUNK
assert _NCHUNK % _NBUF == 0


def _gather_kernel(idx_hbm, table_hbm, out_hbm, idx_v, rows_v, sems_g, sems_o):
    wid = lax.axis_index("s") * _NC + lax.axis_index("c")
    base = wid * _PER_W

    pltpu.sync_copy(idx_hbm.at[pl.ds(base, _PER_W)], idx_v)

    def gather_start(c, b):
        pltpu.async_copy(
            table_hbm.at[idx_v.at[pl.ds(c * _CHUNK, _CHUNK)]],
            rows_v.at[b],
            sems_g.at[b],
        )

    def gather_wait(c, b):
        pltpu.make_async_copy(
            table_hbm.at[idx_v.at[pl.ds(c * _CHUNK, _CHUNK)]],
            rows_v.at[b],
            sems_g.at[b],
        ).wait()

    def out_start(c, b):
        pltpu.async_copy(
            rows_v.at[b], out_hbm.at[pl.ds(base + c * _CHUNK, _CHUNK)], sems_o.at[b]
        )

    def out_wait(c, b):
        pltpu.make_async_copy(
            rows_v.at[b], out_hbm.at[pl.ds(base + c * _CHUNK, _CHUNK)], sems_o.at[b]
        ).wait()

    for b in range(_NBUF):
        gather_start(b, b)

    @pl.loop(0, _NCHUNK - _NBUF, step=_NBUF)
    def _(g):
        for b in range(_NBUF):
            gather_wait(g + b, b)
            out_start(g + b, b)
        for b in range(_NBUF):
            out_wait(g + b, b)
            gather_start(g + _NBUF + b, b)

    last = _NCHUNK - _NBUF
    for b in range(_NBUF):
        gather_wait(last + b, b)
        out_start(last + b, b)
    for b in range(_NBUF):
        out_wait(last + b, b)


@jax.jit
def kernel(tokens, word_embed_weight):
    idx = tokens.reshape(_N).astype(jnp.int32)
    mesh = plsc.VectorSubcoreMesh(core_axis_name="c", subcore_axis_name="s")
    out = pl.kernel(
        _gather_kernel,
        out_type=jax.ShapeDtypeStruct((_N, EMBED_DIM), jnp.float32),
        mesh=mesh,
        scratch_types=[
            pltpu.VMEM((_PER_W,), jnp.int32),
            pltpu.VMEM((_NBUF, _CHUNK, EMBED_DIM), jnp.float32),
            pltpu.SemaphoreType.DMA((_NBUF,)),
            pltpu.SemaphoreType.DMA((_NBUF,)),
        ],
        compiler_params=pltpu.CompilerParams(use_tc_tiling_on_sc=False),
    )(idx, word_embed_weight)
    return out.reshape(B, L, EMBED_DIM)


# speedup vs baseline: 1.2655x; 1.2164x over previous
import jax
import jax.numpy as jnp
from jax import lax
from jax.experimental import pallas as pl
from jax.experimental.pallas import tpu as pltpu
from jax.experimental.pallas import tpu_sc as plsc

B = 4096
L = 200
VOCAB = 1000000
EMBED_DIM = 64
PAD_DIM = 128

_info = plsc.get_sparse_core_info()
_NC = _info.num_cores
_NS = _info.num_subcores
_NW = _NC * _NS

_N = B * L
_PER_W = _N // _NW
_CHUNK = 128
_NBUF = 4
_NCHUNK = _PER_W // _CHUNK
assert _NCHUNK % _NBUF == 0


def _gather_kernel(idx_hbm, table_hbm, out_hbm, idx_v, rows_v, sems_g, sems_o):
    wid = lax.axis_index("s") * _NC + lax.axis_index("c")
    base = wid * _PER_W

    pltpu.sync_copy(idx_hbm.at[pl.ds(base, _PER_W)], idx_v)

    def gather_start(c, b):
        pltpu.async_copy(
            table_hbm.at[idx_v.at[pl.ds(c * _CHUNK, _CHUNK)]],
            rows_v.at[b],
            sems_g.at[b],
        )

    def gather_wait(c, b):
        pltpu.make_async_copy(
            table_hbm.at[idx_v.at[pl.ds(c * _CHUNK, _CHUNK)]],
            rows_v.at[b],
            sems_g.at[b],
        ).wait()

    def out_start(c, b):
        pltpu.async_copy(
            rows_v.at[b, :, pl.ds(0, EMBED_DIM)],
            out_hbm.at[pl.ds(base + c * _CHUNK, _CHUNK), pl.ds(0, EMBED_DIM)],
            sems_o.at[b],
        )

    def out_wait(c, b):
        pltpu.make_async_copy(
            rows_v.at[b, :, pl.ds(0, EMBED_DIM)],
            out_hbm.at[pl.ds(base + c * _CHUNK, _CHUNK), pl.ds(0, EMBED_DIM)],
            sems_o.at[b],
        ).wait()

    for b in range(_NBUF):
        gather_start(b, b)

    @pl.loop(0, _NCHUNK - _NBUF, step=_NBUF)
    def _(g):
        for b in range(_NBUF):
            gather_wait(g + b, b)
            out_start(g + b, b)
        for b in range(_NBUF):
            out_wait(g + b, b)
            gather_start(g + _NBUF + b, b)

    last = _NCHUNK - _NBUF
    for b in range(_NBUF):
        gather_wait(last + b, b)
        out_start(last + b, b)
    for b in range(_NBUF):
        out_wait(last + b, b)


@jax.jit
def kernel(tokens, word_embed_weight):
    idx = tokens.reshape(_N).astype(jnp.int32)
    tpad = jnp.pad(word_embed_weight, ((0, 0), (0, PAD_DIM - EMBED_DIM)))
    mesh = plsc.VectorSubcoreMesh(core_axis_name="c", subcore_axis_name="s")
    out = pl.kernel(
        _gather_kernel,
        out_type=jax.ShapeDtypeStruct((_N, PAD_DIM), jnp.float32),
        mesh=mesh,
        scratch_types=[
            pltpu.VMEM((_PER_W,), jnp.int32),
            pltpu.VMEM((_NBUF, _CHUNK, PAD_DIM), jnp.float32),
            pltpu.SemaphoreType.DMA((_NBUF,)),
            pltpu.SemaphoreType.DMA((_NBUF,)),
        ],
        compiler_params=pltpu.CompilerParams(use_tc_tiling_on_sc=False),
    )(idx, tpad)
    return out[:, :EMBED_DIM].reshape(B, L, EMBED_DIM)
